# baseline (speedup 1.0000x reference)
import functools

import jax
import jax.numpy as jnp
from jax import lax
from jax.experimental import pallas as pl
from jax.experimental.pallas import tpu as pltpu
from jax.experimental.pallas import tpu_sc as plsc

N = 10000
E = 320000
D = 128
DH = D // 2

NC = 2
NS = 16
NW = NC * NS

CHUNK = 80
NCH = E // NS // CHUNK
NPAIR = NCH // 2

N_AL = 10240
SPT = N_AL // NS
RPW = N // NS
ZROWS = 125
YR = 320

_f32 = jnp.float32
_i32 = jnp.int32

_MESH = plsc.VectorSubcoreMesh(
    core_axis_name="c", subcore_axis_name="s", num_cores=NC, num_subcores=NS)

_SC_PARAMS = pltpu.CompilerParams(use_tc_tiling_on_sc=False)


@functools.partial(
    pl.kernel,
    out_type=(
        jax.ShapeDtypeStruct((NC, N, DH), _f32),
        jax.ShapeDtypeStruct((NC, N), _f32),
        jax.ShapeDtypeStruct((N_AL,), _f32),
        jax.ShapeDtypeStruct((N_AL,), _f32),
        jax.ShapeDtypeStruct((N, DH), _f32),
        jax.ShapeDtypeStruct((N, DH), _f32),
    ),
    mesh=_MESH,
    scratch_types=[
        pltpu.VMEM((NCH, CHUNK), _i32),
        pltpu.VMEM((NCH, CHUNK), _i32),
        pltpu.VMEM((CHUNK, DH), _f32),
        pltpu.VMEM((CHUNK, DH), _f32),
        pltpu.VMEM((CHUNK,), _f32),
        pltpu.VMEM((YR, DH), _f32),
        pltpu.VMEM((SPT,), _f32),
        pltpu.SMEM((SPT,), _f32),
        pltpu.VMEM_SHARED((N + 8, DH), _f32),
        pltpu.VMEM_SHARED((N,), _f32),
        pltpu.VMEM_SHARED((N_AL,), _f32),
        pltpu.VMEM_SHARED((N_AL,), _f32),
        pltpu.SemaphoreType.DMA,
        pltpu.SemaphoreType.DMA,
        pltpu.SemaphoreType.DMA,
        pltpu.SemaphoreType.DMA,
    ],
    compiler_params=_SC_PARAMS,
)
def _mega_kernel(src_hbm, dst_hbm, xw_hbm,
                 t_out, u_out, d0_hbm, d1_hbm, y0_hbm, y1_hbm,
                 src_v, dst_v, rows_a, rows_b, vals_v, big_v, work_v,
                 smem_v, t_sh, u_sh, deg_sh, dinv_sh, sem_a, sem_b, sem_u,
                 sem_us):
    c = lax.axis_index("c")
    s = lax.axis_index("s")

    z = jnp.zeros((16,), _f32)

    def zbig_body(i, _):
        for k in range(DH // 16):
            big_v[i, pl.ds(16 * k, 16)] = z
        return 0

    lax.fori_loop(0, YR, zbig_body, 0)

    def zwork_body(i, _):
        work_v[pl.ds(16 * i, 16)] = z
        return 0

    lax.fori_loop(0, SPT // 16, zwork_body, 0)

    for q in range(RPW // ZROWS):
        pltpu.sync_copy(big_v.at[pl.ds(0, ZROWS)],
                        t_sh.at[pl.ds(s * RPW + q * ZROWS, ZROWS)])
    pltpu.sync_copy(work_v, deg_sh.at[pl.ds(s * SPT, SPT)])

    @pl.when(s < NS - 1)
    def _():
        pltpu.sync_copy(work_v, u_sh.at[pl.ds(s * SPT, SPT)])

    @pl.when(s == NS - 1)
    def _():
        pltpu.sync_copy(work_v.at[pl.ds(0, N - (NS - 1) * SPT)],
                        u_sh.at[pl.ds((NS - 1) * SPT, N - (NS - 1) * SPT)])

    one = jnp.full((16,), 1.0, _f32)
    for k in range(CHUNK // 16):
        work_v[pl.ds(16 * k, 16)] = one

    pltpu.sync_copy(src_hbm.at[s], src_v)
    pltpu.sync_copy(dst_hbm.at[s], dst_v)
    plsc.subcore_barrier()

    HK = 5

    def hist_body(g, _):
        for k in range(HK):
            pltpu.async_copy(work_v.at[pl.ds(0, CHUNK)],
                             deg_sh.at[dst_v.at[g * HK + k]], sem_u,
                             add=True)
        for k in range(HK):
            pltpu.make_async_copy(work_v.at[pl.ds(0, CHUNK)],
                                  deg_sh.at[dst_v.at[g * HK + k]],
                                  sem_u).wait()
        return 0

    lax.fori_loop(0, NCH // HK, hist_body, 0)
    plsc.subcore_barrier()

    pltpu.sync_copy(deg_sh.at[pl.ds(s * SPT, SPT)], work_v)

    def newton_body(i, _):
        d = work_v[pl.ds(16 * i, 16)] + 1.0
        y = jnp.full((16,), 1.0, _f32)
        for _ in range(10):
            y = jnp.where(d * y * y > 2.0, y * 0.5, y)
        for _ in range(5):
            y = y * (1.5 - 0.5 * d * y * y)
        work_v[pl.ds(16 * i, 16)] = y
        for j in range(16):
            smem_v[16 * i + j] = y[j]
        return 0

    lax.fori_loop(0, SPT // 16, newton_body, 0)
    pltpu.sync_copy(work_v, dinv_sh.at[pl.ds(s * SPT, SPT)])

    @pl.when(c == 0)
    def _():
        pltpu.sync_copy(work_v, d0_hbm.at[pl.ds(s * SPT, SPT)])

    @pl.when(c == 1)
    def _():
        pltpu.sync_copy(work_v, d1_hbm.at[pl.ds(s * SPT, SPT)])

    def scale_rows(row0, nrows, soff, col0, y_hbm):
        pltpu.sync_copy(xw_hbm.at[pl.ds(row0, nrows), pl.ds(col0, DH)],
                        big_v.at[pl.ds(0, nrows)])

        def row_body(r, _):
            dv = jnp.full((16,), smem_v[soff + r], _f32)
            for k in range(DH // 16):
                big_v[r, pl.ds(16 * k, 16)] = (
                    big_v[r, pl.ds(16 * k, 16)] * dv)
            return 0

        lax.fori_loop(0, nrows, row_body, 0)
        pltpu.sync_copy(big_v.at[pl.ds(0, nrows)],
                        y_hbm.at[pl.ds(row0, nrows)])

    def scale_half(col0, y_hbm):
        @pl.when(s < NS - 1)
        def _():
            scale_rows(s * SPT, YR, 0, col0, y_hbm)
            scale_rows(s * SPT + YR, YR, YR, col0, y_hbm)

        @pl.when(s == NS - 1)
        def _():
            scale_rows((NS - 1) * SPT, YR, 0, col0, y_hbm)
            scale_rows((NS - 1) * SPT + YR, N - (NS - 1) * SPT - YR, YR,
                       col0, y_hbm)

    @pl.when(c == 0)
    def _():
        scale_half(0, y0_hbm)

    @pl.when(c == 1)
    def _():
        scale_half(DH, y1_hbm)

    plsc.subcore_barrier()

    def start_rows(j, buf, sem):
        @pl.when(c == 0)
        def _():
            pltpu.async_copy(y0_hbm.at[src_v.at[j]], buf, sem)

        @pl.when(c == 1)
        def _():
            pltpu.async_copy(y1_hbm.at[src_v.at[j]], buf, sem)

    def wait_rows(j, buf, sem):
        pltpu.make_async_copy(y0_hbm.at[src_v.at[j]], buf, sem).wait()

    def start_vals(j):
        pltpu.async_copy(dinv_sh.at[dst_v.at[j]], vals_v, sem_u)

    start_rows(0, rows_a, sem_a)

    def body(i, _):
        j0 = 2 * i
        j1 = 2 * i + 1
        jm = j0 + c

        @pl.when(i > 0)
        def _():
            pltpu.make_async_copy(vals_v, u_sh.at[src_v.at[jm - 2]],
                                  sem_us).wait()

        start_rows(j1, rows_b, sem_b)
        start_vals(jm)
        wait_rows(j0, rows_a, sem_a)
        pltpu.sync_copy(rows_a, t_sh.at[dst_v.at[j0]], add=True)

        @pl.when(i < NPAIR - 1)
        def _():
            start_rows(j0 + 2, rows_a, sem_a)

        wait_rows(j1, rows_b, sem_b)
        pltpu.sync_copy(rows_b, t_sh.at[dst_v.at[j1]], add=True)

        pltpu.make_async_copy(dinv_sh.at[dst_v.at[jm]], vals_v, sem_u).wait()
        pltpu.async_copy(vals_v, u_sh.at[src_v.at[jm]], sem_us, add=True)
        return 0

    lax.fori_loop(0, NPAIR, body, 0)
    pltpu.make_async_copy(vals_v, u_sh.at[src_v.at[2 * (NPAIR - 1) + c]],
                          sem_us).wait()
    plsc.subcore_barrier()

    pltpu.sync_copy(t_sh.at[pl.ds(s * RPW, RPW)],
                    t_out.at[c, pl.ds(s * RPW, RPW)])

    @pl.when(s == 0)
    def _():
        pltpu.sync_copy(u_sh, u_out.at[c])


def _mm_body(x_ref, w1_ref, xw_ref):
    xw_ref[...] = jnp.dot(x_ref[...], w1_ref[...],
                          preferred_element_type=_f32,
                          precision=lax.Precision.HIGHEST)


def _final_body(t_ref, y0_ref, y1_ref, dinv_ref, dl_ref, u_ref, b1_ref,
                w2_ref, b2_ref, v_ref, out_ref):
    i = pl.program_id(0)
    n_blocks = pl.num_programs(0)
    tp = t_ref[...]
    dv = dinv_ref[...]
    b1 = b1_ref[...]
    m0 = jnp.maximum(dv * tp[0] + y0_ref[...] * dv + b1[:, :DH], 0.0)
    m1 = jnp.maximum(dv * tp[1] + y1_ref[...] * dv + b1[:, DH:], 0.0)
    dl = dl_ref[...][0]
    ul = u_ref[...][:, 0]
    cc = dl * (ul[0] + ul[1] + dl)
    pv0 = jnp.dot(cc, m0, preferred_element_type=_f32,
                  precision=lax.Precision.HIGHEST)
    pv1 = jnp.dot(cc, m1, preferred_element_type=_f32,
                  precision=lax.Precision.HIGHEST)

    @pl.when(i == 0)
    def _():
        v_ref[0] = pv0
        v_ref[1] = pv1

    @pl.when(i > 0)
    def _():
        v_ref[0] = v_ref[0] + pv0
        v_ref[1] = v_ref[1] + pv1

    @pl.when(i == n_blocks - 1)
    def _():
        w2 = w2_ref[...]
        out = (jnp.dot(v_ref[0], w2[:DH, :], preferred_element_type=_f32,
                       precision=lax.Precision.HIGHEST)
               + jnp.dot(v_ref[1], w2[DH:, :], preferred_element_type=_f32,
                         precision=lax.Precision.HIGHEST))
        out_ref[...] = out * (1.0 / N) + b2_ref[...]


def kernel(x, edge_index, W1, b1, W2, b2):
    R = 1000
    G = N // R

    src2 = edge_index[0].reshape(NS, NCH, CHUNK)
    dst2 = edge_index[1].reshape(NS, NCH, CHUNK)

    xw = pl.pallas_call(
        _mm_body,
        grid=(G,),
        in_specs=[
            pl.BlockSpec((R, D), lambda i: (i, 0)),
            pl.BlockSpec((D, D), lambda i: (0, 0)),
        ],
        out_specs=pl.BlockSpec((R, D), lambda i: (i, 0)),
        out_shape=jax.ShapeDtypeStruct((N, D), _f32),
    )(x, W1)

    t_pair, u_pair, d0, _d1, y0, y1 = _mega_kernel(src2, dst2, xw)
    dinv2 = d0[:N].reshape(N, 1)
    dinv_lane = d0[:N].reshape(G, 1, R)
    u_lane = u_pair.reshape(NC, G, 1, R)

    _, out2 = pl.pallas_call(
        _final_body,
        grid=(G,),
        in_specs=[
            pl.BlockSpec((NC, R, DH), lambda i: (0, i, 0)),
            pl.BlockSpec((R, DH), lambda i: (i, 0)),
            pl.BlockSpec((R, DH), lambda i: (i, 0)),
            pl.BlockSpec((R, 1), lambda i: (i, 0)),
            pl.BlockSpec((1, 1, R), lambda i: (i, 0, 0)),
            pl.BlockSpec((NC, 1, 1, R), lambda i: (0, i, 0, 0)),
            pl.BlockSpec((1, D), lambda i: (0, 0)),
            pl.BlockSpec((D, D), lambda i: (0, 0)),
            pl.BlockSpec((1, D), lambda i: (0, 0)),
        ],
        out_specs=[
            pl.BlockSpec((NC, 1, DH), lambda i: (0, 0, 0)),
            pl.BlockSpec((1, D), lambda i: (0, 0)),
        ],
        out_shape=[
            jax.ShapeDtypeStruct((NC, 1, DH), _f32),
            jax.ShapeDtypeStruct((1, D), _f32),
        ],
    )(t_pair, y0, y1, dinv2, dinv_lane, u_lane, b1.reshape(1, D), W2,
      b2.reshape(1, D))

    return out2[0]


# speedup vs baseline: 1.0066x; 1.0066x over previous
import functools

import jax
import jax.numpy as jnp
from jax import lax
from jax.experimental import pallas as pl
from jax.experimental.pallas import tpu as pltpu
from jax.experimental.pallas import tpu_sc as plsc

N = 10000
E = 320000
D = 128
DH = D // 2

NC = 2
NS = 16
NW = NC * NS

CHUNK = 80
NCH = E // NS // CHUNK
NPAIR = NCH // 2

N_AL = 10240
SPT = N_AL // NS
RPW = N // NS
ZROWS = 125
YR = 320

_f32 = jnp.float32
_i32 = jnp.int32

_MESH = plsc.VectorSubcoreMesh(
    core_axis_name="c", subcore_axis_name="s", num_cores=NC, num_subcores=NS)

_SC_PARAMS = pltpu.CompilerParams(use_tc_tiling_on_sc=False)


@functools.partial(
    pl.kernel,
    out_type=(
        jax.ShapeDtypeStruct((NC, N, DH), _f32),
        jax.ShapeDtypeStruct((NC, N), _f32),
        jax.ShapeDtypeStruct((N_AL,), _f32),
        jax.ShapeDtypeStruct((N_AL,), _f32),
        jax.ShapeDtypeStruct((N, DH), _f32),
        jax.ShapeDtypeStruct((N, DH), _f32),
    ),
    mesh=_MESH,
    scratch_types=[
        pltpu.VMEM((NCH, CHUNK), _i32),
        pltpu.VMEM((NCH, CHUNK), _i32),
        pltpu.VMEM((CHUNK, DH), _f32),
        pltpu.VMEM((CHUNK, DH), _f32),
        pltpu.VMEM((CHUNK,), _f32),
        pltpu.VMEM((YR, DH), _f32),
        pltpu.VMEM((SPT,), _f32),
        pltpu.SMEM((SPT,), _f32),
        pltpu.VMEM_SHARED((N + 8, DH), _f32),
        pltpu.VMEM_SHARED((N,), _f32),
        pltpu.VMEM_SHARED((N_AL,), _f32),
        pltpu.VMEM_SHARED((N_AL,), _f32),
        pltpu.SemaphoreType.DMA,
        pltpu.SemaphoreType.DMA,
        pltpu.SemaphoreType.DMA,
    ],
    compiler_params=_SC_PARAMS,
)
def _mega_kernel(src_hbm, dst_hbm, xw_hbm,
                 t_out, u_out, d0_hbm, d1_hbm, y0_hbm, y1_hbm,
                 src_v, dst_v, rows_a, rows_b, vals_v, big_v, work_v,
                 smem_v, t_sh, u_sh, deg_sh, dinv_sh, sem_a, sem_b, sem_u):
    c = lax.axis_index("c")
    s = lax.axis_index("s")

    z = jnp.zeros((16,), _f32)

    def zbig_body(i, _):
        for k in range(DH // 16):
            big_v[i, pl.ds(16 * k, 16)] = z
        return 0

    lax.fori_loop(0, YR, zbig_body, 0)

    def zwork_body(i, _):
        work_v[pl.ds(16 * i, 16)] = z
        return 0

    lax.fori_loop(0, SPT // 16, zwork_body, 0)

    for q in range(RPW // ZROWS):
        pltpu.sync_copy(big_v.at[pl.ds(0, ZROWS)],
                        t_sh.at[pl.ds(s * RPW + q * ZROWS, ZROWS)])
    pltpu.sync_copy(work_v, deg_sh.at[pl.ds(s * SPT, SPT)])

    @pl.when(s < NS - 1)
    def _():
        pltpu.sync_copy(work_v, u_sh.at[pl.ds(s * SPT, SPT)])

    @pl.when(s == NS - 1)
    def _():
        pltpu.sync_copy(work_v.at[pl.ds(0, N - (NS - 1) * SPT)],
                        u_sh.at[pl.ds((NS - 1) * SPT, N - (NS - 1) * SPT)])

    one = jnp.full((16,), 1.0, _f32)
    for k in range(CHUNK // 16):
        work_v[pl.ds(16 * k, 16)] = one

    pltpu.sync_copy(src_hbm.at[s], src_v)
    pltpu.sync_copy(dst_hbm.at[s], dst_v)
    plsc.subcore_barrier()

    HK = 10

    def hist_body(g, _):
        for k in range(HK):
            pltpu.async_copy(work_v.at[pl.ds(0, CHUNK)],
                             deg_sh.at[dst_v.at[g * HK + k]], sem_u,
                             add=True)
        for k in range(HK):
            pltpu.make_async_copy(work_v.at[pl.ds(0, CHUNK)],
                                  deg_sh.at[dst_v.at[g * HK + k]],
                                  sem_u).wait()
        return 0

    lax.fori_loop(0, NCH // HK, hist_body, 0)
    plsc.subcore_barrier()

    pltpu.sync_copy(deg_sh.at[pl.ds(s * SPT, SPT)], work_v)

    def newton_body(i, _):
        d = work_v[pl.ds(16 * i, 16)] + 1.0
        y = jnp.full((16,), 1.0, _f32)
        for _ in range(10):
            y = jnp.where(d * y * y > 2.0, y * 0.5, y)
        for _ in range(5):
            y = y * (1.5 - 0.5 * d * y * y)
        work_v[pl.ds(16 * i, 16)] = y
        for j in range(16):
            smem_v[16 * i + j] = y[j]
        return 0

    lax.fori_loop(0, SPT // 16, newton_body, 0)
    pltpu.sync_copy(work_v, dinv_sh.at[pl.ds(s * SPT, SPT)])

    @pl.when(c == 0)
    def _():
        pltpu.sync_copy(work_v, d0_hbm.at[pl.ds(s * SPT, SPT)])

    @pl.when(c == 1)
    def _():
        pltpu.sync_copy(work_v, d1_hbm.at[pl.ds(s * SPT, SPT)])

    def scale_rows(row0, nrows, soff, col0, y_hbm):
        pltpu.sync_copy(xw_hbm.at[pl.ds(row0, nrows), pl.ds(col0, DH)],
                        big_v.at[pl.ds(0, nrows)])

        def row_body(r, _):
            dv = jnp.full((16,), smem_v[soff + r], _f32)
            for k in range(DH // 16):
                big_v[r, pl.ds(16 * k, 16)] = (
                    big_v[r, pl.ds(16 * k, 16)] * dv)
            return 0

        lax.fori_loop(0, nrows, row_body, 0)
        pltpu.sync_copy(big_v.at[pl.ds(0, nrows)],
                        y_hbm.at[pl.ds(row0, nrows)])

    def scale_half(col0, y_hbm):
        @pl.when(s < NS - 1)
        def _():
            scale_rows(s * SPT, YR, 0, col0, y_hbm)
            scale_rows(s * SPT + YR, YR, YR, col0, y_hbm)

        @pl.when(s == NS - 1)
        def _():
            scale_rows((NS - 1) * SPT, YR, 0, col0, y_hbm)
            scale_rows((NS - 1) * SPT + YR, N - (NS - 1) * SPT - YR, YR,
                       col0, y_hbm)

    @pl.when(c == 0)
    def _():
        scale_half(0, y0_hbm)

    @pl.when(c == 1)
    def _():
        scale_half(DH, y1_hbm)

    plsc.subcore_barrier()

    def start_rows(j, buf, sem):
        @pl.when(c == 0)
        def _():
            pltpu.async_copy(y0_hbm.at[src_v.at[j]], buf, sem)

        @pl.when(c == 1)
        def _():
            pltpu.async_copy(y1_hbm.at[src_v.at[j]], buf, sem)

    def wait_rows(j, buf, sem):
        pltpu.make_async_copy(y0_hbm.at[src_v.at[j]], buf, sem).wait()

    def start_vals(j):
        pltpu.async_copy(dinv_sh.at[dst_v.at[j]], vals_v, sem_u)

    start_rows(0, rows_a, sem_a)

    def body(i, _):
        j0 = 2 * i
        j1 = 2 * i + 1
        jm = j0 + c
        start_rows(j1, rows_b, sem_b)
        start_vals(jm)
        wait_rows(j0, rows_a, sem_a)
        pltpu.sync_copy(rows_a, t_sh.at[dst_v.at[j0]], add=True)

        @pl.when(i < NPAIR - 1)
        def _():
            start_rows(j0 + 2, rows_a, sem_a)

        wait_rows(j1, rows_b, sem_b)
        pltpu.sync_copy(rows_b, t_sh.at[dst_v.at[j1]], add=True)

        pltpu.make_async_copy(dinv_sh.at[dst_v.at[jm]], vals_v, sem_u).wait()
        pltpu.sync_copy(vals_v, u_sh.at[src_v.at[jm]], add=True)
        return 0

    lax.fori_loop(0, NPAIR, body, 0)
    plsc.subcore_barrier()

    pltpu.sync_copy(t_sh.at[pl.ds(s * RPW, RPW)],
                    t_out.at[c, pl.ds(s * RPW, RPW)])

    @pl.when(s == 0)
    def _():
        pltpu.sync_copy(u_sh, u_out.at[c])


def _mm_body(x_ref, w1_ref, xw_ref):
    xw_ref[...] = jnp.dot(x_ref[...], w1_ref[...],
                          preferred_element_type=_f32,
                          precision=lax.Precision.HIGHEST)


def _final_body(t_ref, y0_ref, y1_ref, dinv_ref, dl_ref, u_ref, b1_ref,
                w2_ref, b2_ref, v_ref, out_ref):
    i = pl.program_id(0)
    n_blocks = pl.num_programs(0)
    tp = t_ref[...]
    dv = dinv_ref[...]
    b1 = b1_ref[...]
    m0 = jnp.maximum(dv * tp[0] + y0_ref[...] * dv + b1[:, :DH], 0.0)
    m1 = jnp.maximum(dv * tp[1] + y1_ref[...] * dv + b1[:, DH:], 0.0)
    dl = dl_ref[...][0]
    ul = u_ref[...][:, 0]
    cc = dl * (ul[0] + ul[1] + dl)
    pv0 = jnp.dot(cc, m0, preferred_element_type=_f32,
                  precision=lax.Precision.HIGHEST)
    pv1 = jnp.dot(cc, m1, preferred_element_type=_f32,
                  precision=lax.Precision.HIGHEST)

    @pl.when(i == 0)
    def _():
        v_ref[0] = pv0
        v_ref[1] = pv1

    @pl.when(i > 0)
    def _():
        v_ref[0] = v_ref[0] + pv0
        v_ref[1] = v_ref[1] + pv1

    @pl.when(i == n_blocks - 1)
    def _():
        w2 = w2_ref[...]
        out = (jnp.dot(v_ref[0], w2[:DH, :], preferred_element_type=_f32,
                       precision=lax.Precision.HIGHEST)
               + jnp.dot(v_ref[1], w2[DH:, :], preferred_element_type=_f32,
                         precision=lax.Precision.HIGHEST))
        out_ref[...] = out * (1.0 / N) + b2_ref[...]


def kernel(x, edge_index, W1, b1, W2, b2):
    R = 1000
    G = N // R

    src2 = edge_index[0].reshape(NS, NCH, CHUNK)
    dst2 = edge_index[1].reshape(NS, NCH, CHUNK)

    xw = pl.pallas_call(
        _mm_body,
        grid=(G,),
        in_specs=[
            pl.BlockSpec((R, D), lambda i: (i, 0)),
            pl.BlockSpec((D, D), lambda i: (0, 0)),
        ],
        out_specs=pl.BlockSpec((R, D), lambda i: (i, 0)),
        out_shape=jax.ShapeDtypeStruct((N, D), _f32),
    )(x, W1)

    t_pair, u_pair, d0, _d1, y0, y1 = _mega_kernel(src2, dst2, xw)
    dinv2 = d0[:N].reshape(N, 1)
    dinv_lane = d0[:N].reshape(G, 1, R)
    u_lane = u_pair.reshape(NC, G, 1, R)

    _, out2 = pl.pallas_call(
        _final_body,
        grid=(G,),
        in_specs=[
            pl.BlockSpec((NC, R, DH), lambda i: (0, i, 0)),
            pl.BlockSpec((R, DH), lambda i: (i, 0)),
            pl.BlockSpec((R, DH), lambda i: (i, 0)),
            pl.BlockSpec((R, 1), lambda i: (i, 0)),
            pl.BlockSpec((1, 1, R), lambda i: (i, 0, 0)),
            pl.BlockSpec((NC, 1, 1, R), lambda i: (0, i, 0, 0)),
            pl.BlockSpec((1, D), lambda i: (0, 0)),
            pl.BlockSpec((D, D), lambda i: (0, 0)),
            pl.BlockSpec((1, D), lambda i: (0, 0)),
        ],
        out_specs=[
            pl.BlockSpec((NC, 1, DH), lambda i: (0, 0, 0)),
            pl.BlockSpec((1, D), lambda i: (0, 0)),
        ],
        out_shape=[
            jax.ShapeDtypeStruct((NC, 1, DH), _f32),
            jax.ShapeDtypeStruct((1, D), _f32),
        ],
    )(t_pair, y0, y1, dinv2, dinv_lane, u_lane, b1.reshape(1, D), W2,
      b2.reshape(1, D))

    return out2[0]
